# baseline (speedup 1.0000x reference)
import functools

import jax
import jax.numpy as jnp
from jax import lax
from jax.experimental import pallas as pl
from jax.experimental.pallas import tpu as pltpu
from jax.experimental.pallas import tpu_sc as plsc

_B = 112
_CHB = 9
_LANES = 16


def _mm_body(x_ref, w_ref, o_ref):
    o_ref[...] = jnp.dot(x_ref[...], w_ref[...], preferred_element_type=jnp.float32)


def _combine_body(n_nodes, p_ref, o_ref):
    o_ref[...] = jnp.maximum(p_ref[0, :n_nodes, :] + p_ref[1, :n_nodes, :], 0.0)


def _make_sc_call(n_nodes, d, nb, nc, ns):
    nw = nc * ns
    n_groups = _B // _LANES
    n_sub = d // _LANES
    n_chunks_e = nb // _CHB
    rows_per_tile = -(-n_nodes // (ns * 8)) * 8
    n_rows = rows_per_tile * ns
    chunks = [_B] * (rows_per_tile // _B)
    if rows_per_tile % _B:
        chunks.append(rows_per_tile % _B)

    mesh = plsc.VectorSubcoreMesh(core_axis_name="c", subcore_axis_name="s")

    @functools.partial(
        pl.kernel,
        mesh=mesh,
        out_type=jax.ShapeDtypeStruct((nc, n_rows, d), jnp.float32),
        scratch_types=[
            pltpu.VMEM_SHARED((n_rows, d), jnp.float32),
            pltpu.VMEM((_CHB * _B,), jnp.int32),
            pltpu.VMEM((_CHB * _B,), jnp.int32),
            pltpu.VMEM((_B,), jnp.int32),
            pltpu.VMEM((_B,), jnp.int32),
            pltpu.VMEM((_B,), jnp.int32),
            pltpu.VMEM((_B,), jnp.float32),
            pltpu.VMEM((_B,), jnp.float32),
            pltpu.VMEM((_B,), jnp.float32),
            pltpu.VMEM((_B, d), jnp.float32),
            pltpu.VMEM((_B, d), jnp.float32),
            pltpu.VMEM((_B, d), jnp.float32),
            pltpu.SemaphoreType.DMA,
            pltpu.SemaphoreType.DMA,
            pltpu.SemaphoreType.DMA,
            pltpu.SemaphoreType.DMA,
            pltpu.SemaphoreType.DMA,
            pltpu.SemaphoreType.DMA,
            pltpu.SemaphoreType.DMA,
            pltpu.SemaphoreType.DMA,
            pltpu.SemaphoreType.DMA,
            pltpu.SemaphoreType.DMA,
            pltpu.SemaphoreType.DMA,
        ],
    )
    def sc_call(h_hbm, src_hbm, dst_hbm, adj_hbm, out_hbm,
                acc, srcc0, srcc1, dstp0, dstp1, dstp2, adjp0, adjp1, adjp2,
                rows0, rows1, rows2,
                sem_c0, sem_c1, sem_i0, sem_i1, sem_i2,
                sem_r0, sem_r1, sem_r2, sem_s0, sem_s1, sem_s2):
        cid = lax.axis_index("c")
        sid = lax.axis_index("s")
        wid = sid * nc + cid
        ebase = wid * nb * _B

        src_c = (srcc0, srcc1)
        dst_b = (dstp0, dstp1, dstp2)
        adj_b = (adjp0, adjp1, adjp2)
        rows_b = (rows0, rows1, rows2)
        sem_c = (sem_c0, sem_c1)
        sem_i = (sem_i0, sem_i1, sem_i2)
        sem_r = (sem_r0, sem_r1, sem_r2)
        sem_s = (sem_s0, sem_s1, sem_s2)

        def chunk_start(c, P):
            pltpu.make_async_copy(
                src_hbm.at[pl.ds(ebase + c * _CHB * _B, _CHB * _B)],
                src_c[P], sem_c[P]).start()

        def chunk_wait(P):
            pltpu.make_async_copy(
                src_hbm.at[pl.ds(ebase, _CHB * _B)], src_c[P], sem_c[P]).wait()

        def idx_start(b, p):
            pltpu.make_async_copy(
                dst_hbm.at[pl.ds(ebase + b * _B, _B)], dst_b[p], sem_i[p]).start()
            pltpu.make_async_copy(
                adj_hbm.at[pl.ds(ebase + b * _B, _B)], adj_b[p], sem_i[p]).start()

        def idx_wait(p):
            pltpu.make_async_copy(
                dst_hbm.at[pl.ds(ebase, _B)], dst_b[p], sem_i[p]).wait()
            pltpu.make_async_copy(
                adj_hbm.at[pl.ds(ebase, _B)], adj_b[p], sem_i[p]).wait()

        def gather_start(P, jj, p):
            pltpu.make_async_copy(
                h_hbm.at[src_c[P].at[pl.ds(jj * _B, _B)]],
                rows_b[p], sem_r[p]).start()

        def gather_wait(p):
            pltpu.make_async_copy(
                h_hbm.at[src_c[0].at[pl.ds(0, _B)]], rows_b[p], sem_r[p]).wait()

        def scatter_start(p):
            pltpu.make_async_copy(
                rows_b[p], acc.at[dst_b[p]], sem_s[p]).start(add=True)

        def scatter_wait(p):
            pltpu.make_async_copy(
                rows_b[p], acc.at[dst_b[p]], sem_s[p]).wait()

        def scale(p):
            buf = rows_b[p]
            adj = adj_b[p]

            def grp(g, _):
                av = adj[pl.ds(g * _LANES, _LANES)]
                for j in range(_LANES):
                    s = jnp.full((_LANES,), av[j], jnp.float32)
                    r = g * _LANES + j
                    for cch in range(n_sub):
                        sl = pl.ds(cch * _LANES, _LANES)
                        buf[r, sl] = buf[r, sl] * s
                return 0

            lax.fori_loop(0, n_groups, grp, 0)

        zeros16 = jnp.zeros((_LANES,), jnp.float32)

        def zero_row(r, _):
            for cch in range(n_sub):
                rows0[r, pl.ds(cch * _LANES, _LANES)] = zeros16
            return 0

        lax.fori_loop(0, _B, zero_row, 0)
        for k, ch in enumerate(chunks):
            pltpu.sync_copy(
                rows0.at[pl.ds(0, ch)],
                acc.at[pl.ds(sid * rows_per_tile + k * _B, ch)])

        chunk_start(0, 0)
        chunk_start(1, 1)
        idx_start(0, 0)
        idx_start(1, 1)
        chunk_wait(0)
        gather_start(0, 0, 0)
        gather_start(0, 1, 1)
        plsc.subcore_barrier()

        def process_chunk(c, P, PN):
            for j in range(_CHB):
                p = j % 3
                pm1 = (p + 2) % 3
                b = c * _CHB + j
                gather_wait(p)
                idx_wait(p)
                scale(p)
                scatter_start(p)

                @pl.when(b > 0)
                def _():
                    scatter_wait(pm1)

                if j == _CHB - 2:
                    chunk_wait(PN)
                if j < _CHB - 2:
                    gather_start(P, j + 2, pm1)
                else:
                    gather_start(PN, j + 2 - _CHB, pm1)
                idx_start(lax.rem(b + 2, nb), pm1)
            chunk_start(lax.rem(c + 2, n_chunks_e), P)

        def outer(q, _):
            process_chunk(2 * q, 0, 1)
            process_chunk(2 * q + 1, 1, 0)
            return 0

        lax.fori_loop(0, n_chunks_e // 2, outer, 0)
        scatter_wait((nb - 1) % 3)
        gather_wait(0)
        gather_wait(1)
        idx_wait(0)
        idx_wait(1)
        chunk_wait(1)
        plsc.subcore_barrier()

        for k, ch in enumerate(chunks):
            r0 = sid * rows_per_tile + k * _B
            pltpu.sync_copy(acc.at[pl.ds(r0, ch)], out_hbm.at[cid, pl.ds(r0, ch)])

    return sc_call


def kernel(x, edge_index, adj_vals, weight):
    n_nodes, d_in = x.shape
    d_out = weight.shape[1]
    e = adj_vals.shape[0]

    info = plsc.get_sparse_core_info()
    nc, ns = info.num_cores, info.num_subcores
    nw = nc * ns

    align = 2 * _CHB
    nb = -(-e // (nw * _B * align)) * align
    e_slots = nw * nb * _B
    pad_idx = jnp.arange(e_slots - e, dtype=jnp.int32) % n_nodes

    def stage(a, fill):
        return jnp.concatenate([a, fill])

    src = stage(edge_index[0].astype(jnp.int32), pad_idx)
    dst = stage(edge_index[1].astype(jnp.int32), pad_idx)
    adj = stage(adj_vals, jnp.zeros((e_slots - e,), jnp.float32))

    h = pl.pallas_call(
        _mm_body,
        out_shape=jax.ShapeDtypeStruct((n_nodes, d_out), jnp.float32),
    )(x, weight)

    sc_call = _make_sc_call(n_nodes, d_out, nb, nc, ns)
    partials = sc_call(h, src, dst, adj)

    out = pl.pallas_call(
        functools.partial(_combine_body, n_nodes),
        out_shape=jax.ShapeDtypeStruct((n_nodes, d_out), jnp.float32),
    )(partials)
    return out


# speedup vs baseline: 1.0017x; 1.0017x over previous
import functools

import jax
import jax.numpy as jnp
from jax import lax
from jax.experimental import pallas as pl
from jax.experimental.pallas import tpu as pltpu
from jax.experimental.pallas import tpu_sc as plsc

_B = 112
_CHB = 9
_LANES = 16


def _mm_body(x_ref, w_ref, o_ref):
    o_ref[...] = jnp.dot(x_ref[...], w_ref[...], preferred_element_type=jnp.float32)


def _combine_body(n_nodes, p_ref, o_ref):
    o_ref[...] = jnp.maximum(p_ref[0, :n_nodes, :] + p_ref[1, :n_nodes, :], 0.0)


def _make_sc_call(n_nodes, d, nb, nc, ns):
    nw = nc * ns
    n_groups = _B // _LANES
    n_sub = d // _LANES
    n_chunks_e = nb // _CHB
    rows_per_tile = -(-n_nodes // (ns * 8)) * 8
    n_rows = rows_per_tile * ns
    chunks = [_B] * (rows_per_tile // _B)
    if rows_per_tile % _B:
        chunks.append(rows_per_tile % _B)

    mesh = plsc.VectorSubcoreMesh(core_axis_name="c", subcore_axis_name="s")

    @functools.partial(
        pl.kernel,
        mesh=mesh,
        out_type=jax.ShapeDtypeStruct((nc, n_rows, d), jnp.float32),
        scratch_types=[
            pltpu.VMEM_SHARED((n_rows, d), jnp.float32),
            pltpu.VMEM((_CHB * _B,), jnp.int32),
            pltpu.VMEM((_CHB * _B,), jnp.int32),
            pltpu.VMEM((_CHB * _B,), jnp.float32),
            pltpu.VMEM((_CHB * _B,), jnp.float32),
            pltpu.VMEM((_B,), jnp.int32),
            pltpu.VMEM((_B,), jnp.int32),
            pltpu.VMEM((_B,), jnp.int32),
            pltpu.VMEM((_B, d), jnp.float32),
            pltpu.VMEM((_B, d), jnp.float32),
            pltpu.VMEM((_B, d), jnp.float32),
            pltpu.SemaphoreType.DMA,
            pltpu.SemaphoreType.DMA,
            pltpu.SemaphoreType.DMA,
            pltpu.SemaphoreType.DMA,
            pltpu.SemaphoreType.DMA,
            pltpu.SemaphoreType.DMA,
            pltpu.SemaphoreType.DMA,
            pltpu.SemaphoreType.DMA,
            pltpu.SemaphoreType.DMA,
            pltpu.SemaphoreType.DMA,
            pltpu.SemaphoreType.DMA,
        ],
    )
    def sc_call(h_hbm, src_hbm, adj_hbm, dst_hbm, out_hbm,
                acc, srcc0, srcc1, adjc0, adjc1, dstp0, dstp1, dstp2,
                rows0, rows1, rows2,
                sem_c0, sem_c1, sem_i0, sem_i1, sem_i2,
                sem_r0, sem_r1, sem_r2, sem_s0, sem_s1, sem_s2):
        cid = lax.axis_index("c")
        sid = lax.axis_index("s")
        wid = sid * nc + cid
        ebase = wid * nb * _B
        clen = _CHB * _B

        src_c = (srcc0, srcc1)
        adj_c = (adjc0, adjc1)
        dst_b = (dstp0, dstp1, dstp2)
        rows_b = (rows0, rows1, rows2)
        sem_c = (sem_c0, sem_c1)
        sem_i = (sem_i0, sem_i1, sem_i2)
        sem_r = (sem_r0, sem_r1, sem_r2)
        sem_s = (sem_s0, sem_s1, sem_s2)

        def chunk_start(c, P):
            pltpu.make_async_copy(
                src_hbm.at[pl.ds(ebase + c * clen, clen)],
                src_c[P], sem_c[P]).start()
            pltpu.make_async_copy(
                adj_hbm.at[pl.ds(ebase + c * clen, clen)],
                adj_c[P], sem_c[P]).start()

        def chunk_wait(P):
            pltpu.make_async_copy(
                src_hbm.at[pl.ds(0, clen)], src_c[P], sem_c[P]).wait()
            pltpu.make_async_copy(
                adj_hbm.at[pl.ds(0, clen)], adj_c[P], sem_c[P]).wait()

        def idx_start(b, p):
            pltpu.make_async_copy(
                dst_hbm.at[pl.ds(ebase + b * _B, _B)], dst_b[p], sem_i[p]).start()

        def idx_wait(p):
            pltpu.make_async_copy(
                dst_hbm.at[pl.ds(ebase, _B)], dst_b[p], sem_i[p]).wait()

        def gather_start(P, jj, p):
            pltpu.make_async_copy(
                h_hbm.at[src_c[P].at[pl.ds(jj * _B, _B)]],
                rows_b[p], sem_r[p]).start()

        def gather_wait(p):
            pltpu.make_async_copy(
                h_hbm.at[src_c[0].at[pl.ds(0, _B)]], rows_b[p], sem_r[p]).wait()

        def scatter_start(p):
            pltpu.make_async_copy(
                rows_b[p], acc.at[dst_b[p]], sem_s[p]).start(add=True)

        def scatter_wait(p):
            pltpu.make_async_copy(
                rows_b[p], acc.at[dst_b[p]], sem_s[p]).wait()

        def scale(p, P, jj):
            buf = rows_b[p]
            chunk = adj_c[P]
            abase = jj * _B

            def grp(g, _):
                av = chunk[pl.ds(abase + g * _LANES, _LANES)]
                for j in range(_LANES):
                    s = jnp.full((_LANES,), av[j], jnp.float32)
                    r = g * _LANES + j
                    for cch in range(n_sub):
                        sl = pl.ds(cch * _LANES, _LANES)
                        buf[r, sl] = buf[r, sl] * s
                return 0

            lax.fori_loop(0, n_groups, grp, 0)

        zeros16 = jnp.zeros((_LANES,), jnp.float32)

        def zero_row(r, _):
            for cch in range(n_sub):
                rows0[r, pl.ds(cch * _LANES, _LANES)] = zeros16
            return 0

        lax.fori_loop(0, _B, zero_row, 0)
        for k, ch in enumerate(chunks):
            pltpu.sync_copy(
                rows0.at[pl.ds(0, ch)],
                acc.at[pl.ds(sid * rows_per_tile + k * _B, ch)])

        chunk_start(0, 0)
        chunk_start(1, 1)
        idx_start(0, 0)
        idx_start(1, 1)
        chunk_wait(0)
        gather_start(0, 0, 0)
        gather_start(0, 1, 1)
        plsc.subcore_barrier()

        def process_chunk(c, P, PN):
            for j in range(_CHB):
                p = j % 3
                pm1 = (p + 2) % 3
                b = c * _CHB + j
                gather_wait(p)
                idx_wait(p)
                scale(p, P, j)
                scatter_start(p)

                @pl.when(b > 0)
                def _():
                    scatter_wait(pm1)

                if j == _CHB - 2:
                    chunk_wait(PN)
                if j < _CHB - 2:
                    gather_start(P, j + 2, pm1)
                else:
                    gather_start(PN, j + 2 - _CHB, pm1)
                idx_start(lax.rem(b + 2, nb), pm1)
            chunk_start(lax.rem(c + 2, n_chunks_e), P)

        def outer(q, _):
            process_chunk(2 * q, 0, 1)
            process_chunk(2 * q + 1, 1, 0)
            return 0

        lax.fori_loop(0, n_chunks_e // 2, outer, 0)
        scatter_wait((nb - 1) % 3)
        gather_wait(0)
        gather_wait(1)
        idx_wait(0)
        idx_wait(1)
        chunk_wait(1)
        plsc.subcore_barrier()

        for k, ch in enumerate(chunks):
            r0 = sid * rows_per_tile + k * _B
            pltpu.sync_copy(acc.at[pl.ds(r0, ch)], out_hbm.at[cid, pl.ds(r0, ch)])

    return sc_call


def kernel(x, edge_index, adj_vals, weight):
    n_nodes, d_in = x.shape
    d_out = weight.shape[1]
    e = adj_vals.shape[0]

    info = plsc.get_sparse_core_info()
    nc, ns = info.num_cores, info.num_subcores
    nw = nc * ns

    align = 2 * _CHB
    nb = -(-e // (nw * _B * align)) * align
    e_slots = nw * nb * _B
    pad_idx = jnp.arange(e_slots - e, dtype=jnp.int32) % n_nodes

    def stage(a, fill):
        return jnp.concatenate([a, fill])

    src = stage(edge_index[0].astype(jnp.int32), pad_idx)
    dst = stage(edge_index[1].astype(jnp.int32), pad_idx)
    adj = stage(adj_vals, jnp.zeros((e_slots - e,), jnp.float32))

    h = pl.pallas_call(
        _mm_body,
        out_shape=jax.ShapeDtypeStruct((n_nodes, d_out), jnp.float32),
    )(x, weight)

    sc_call = _make_sc_call(n_nodes, d_out, nb, nc, ns)
    partials = sc_call(h, src, adj, dst)

    out = pl.pallas_call(
        functools.partial(_combine_body, n_nodes),
        out_shape=jax.ShapeDtypeStruct((n_nodes, d_out), jnp.float32),
    )(partials)
    return out
